# Initial kernel scaffold; baseline (speedup 1.0000x reference)
import jax
import jax.numpy as jnp
from jax.experimental import pallas as pl


def kernel(hidden_states, w1, b1, w2, b2, dropout_key):
    raise NotImplementedError("write your pallas kernel here")


# speedup vs baseline: 1.1575x; 1.1575x over previous
import jax
import jax.numpy as jnp
from jax.experimental import pallas as pl
from jax.experimental.pallas import tpu as pltpu

_LANE = 128
_ROT_A = (13, 15, 26, 6)
_ROT_B = (17, 29, 16, 24)
_THREEFRY_C = 0x1BD11BDA
_KEEP_THRESHOLD = int(round(0.1 * 2.0 ** 32))
_KEEP_SCALE = 1.0 / (1.0 - 0.1)


def _round_up(x, m):
    return (x + m - 1) // m * m


def _rotl(x, r):
    return (x << jnp.uint32(r)) | (x >> jnp.uint32(32 - r))


def _threefry2x32(k0, k1, x0, x1):
    ks2 = k0 ^ k1 ^ jnp.uint32(_THREEFRY_C)
    ks = (k0, k1, ks2)
    x0 = x0 + k0
    x1 = x1 + k1
    for i in range(5):
        for r in (_ROT_A if i % 2 == 0 else _ROT_B):
            x0 = x0 + x1
            x1 = _rotl(x1, r)
            x1 = x0 ^ x1
        x0 = x0 + ks[(i + 1) % 3]
        x1 = x1 + ks[(i + 2) % 3] + jnp.uint32(i + 1)
    return x0, x1


def _random_bits(k0, k1, idx):
    zero = jnp.zeros_like(idx)
    a, b = _threefry2x32(k0, k1, zero, idx)
    return a ^ b


def _qa_kernel(keys_ref, x_ref, w1_ref, w2_ref, bias_ref, o_ref, *, tm, h, hp):
    i = pl.program_id(0)
    row = jax.lax.broadcasted_iota(jnp.int32, (tm, h), 0)
    col = jax.lax.broadcasted_iota(jnp.int32, (tm, h), 1)
    idx = ((i * tm + row) * hp + col).astype(jnp.uint32)

    bits1 = _random_bits(keys_ref[0], keys_ref[1], idx)
    bits2 = _random_bits(keys_ref[2], keys_ref[3], idx)

    thr = jnp.uint32(_KEEP_THRESHOLD)
    xs = x_ref[...] * _KEEP_SCALE
    x1 = jnp.where(bits1 >= thr, xs, 0.0)
    x2 = jnp.where(bits2 >= thr, xs, 0.0)

    o = jnp.dot(x1, w1_ref[...], preferred_element_type=jnp.float32)
    o += jnp.dot(x2, w2_ref[...], preferred_element_type=jnp.float32)
    o_ref[...] = o + bias_ref[...]


def kernel(hidden_states, w1, b1, w2, b2, dropout_key):
    B, S, H = hidden_states.shape
    nl = w1.shape[1]
    M = B * S
    x = hidden_states.reshape(M, H).astype(jnp.float32)

    TM = min(256, _round_up(M, 8))
    Mp = _round_up(M, TM)
    Hp = _round_up(H, _LANE)
    if (Mp, Hp) != (M, H):
        x = jnp.zeros((Mp, Hp), jnp.float32).at[:M, :H].set(x)

    NP = _LANE
    w1p = jnp.zeros((Hp, NP), jnp.float32).at[:H, :nl].set(w1.astype(jnp.float32))
    w2p = jnp.zeros((Hp, NP), jnp.float32).at[:H, nl:2 * nl].set(w2.astype(jnp.float32))
    bias = jnp.zeros((1, NP), jnp.float32)
    bias = bias.at[0, :nl].set(b1.astype(jnp.float32))
    bias = bias.at[0, nl:2 * nl].set(b2.astype(jnp.float32))

    key = jax.random.wrap_key_data(dropout_key)
    k1, k2 = jax.random.split(key)
    keys4 = jnp.concatenate(
        [jax.random.key_data(k1), jax.random.key_data(k2)]).astype(jnp.uint32)

    grid = (Mp // TM,)
    o = pl.pallas_call(
        lambda *a: _qa_kernel(*a, tm=TM, h=Hp, hp=Hp),
        out_shape=jax.ShapeDtypeStruct((Mp, NP), jnp.float32),
        grid=grid,
        in_specs=[
            pl.BlockSpec(memory_space=pltpu.SMEM),
            pl.BlockSpec((TM, Hp), lambda i: (i, 0)),
            pl.BlockSpec((Hp, NP), lambda i: (0, 0)),
            pl.BlockSpec((Hp, NP), lambda i: (0, 0)),
            pl.BlockSpec((1, NP), lambda i: (0, 0)),
        ],
        out_specs=pl.BlockSpec((TM, NP), lambda i: (i, 0)),
        compiler_params=pltpu.CompilerParams(
            dimension_semantics=("parallel",),
            vmem_limit_bytes=48 * 1024 * 1024,
        ),
    )(keys4, x, w1p, w2p, bias)

    start_logits = o[:M, :nl].reshape(B, S, nl)
    end_logits = o[:M, nl:2 * nl].reshape(B, S, nl)
    return start_logits, end_logits
